# Initial kernel scaffold; baseline (speedup 1.0000x reference)
import jax
import jax.numpy as jnp
from jax.experimental import pallas as pl


def kernel(items, A, B, t):
    raise NotImplementedError("write your pallas kernel here")


# speedup vs baseline: 2.5869x; 2.5869x over previous
import functools

import jax
import jax.numpy as jnp
from jax import lax
from jax.experimental import pallas as pl
from jax.experimental.pallas import tpu as pltpu
from jax.experimental.pallas import tpu_sc as plsc

ITEM_COUNT = 16384
ALPHA = 0.0001
NC = 2
NS = 16
L = 16
NW = NC * NS
PER_W = ITEM_COUNT // NW
ROW = 128
ROWS = PER_W // ROW

_LN2 = 0.6931471805599453
_SQRT2 = 1.4142135623730951

_mesh = plsc.VectorSubcoreMesh(
    core_axis_name="c", subcore_axis_name="s", num_cores=NC, num_subcores=NS
)


@functools.partial(
    pl.kernel,
    out_type=jax.ShapeDtypeStruct((ITEM_COUNT,), jnp.float32),
    mesh=_mesh,
    scratch_types=[
        pltpu.VMEM((ROWS, ROW), jnp.int32),
        pltpu.VMEM((ROWS, ROW), jnp.float32),
        pltpu.VMEM((ROWS, ROW), jnp.float32),
        pltpu.VMEM((ROWS, ROW), jnp.float32),
        pltpu.VMEM((ROWS, ROW), jnp.float32),
        pltpu.VMEM((ROW,), jnp.float32),
        pltpu.VMEM((L,), jnp.float32),
        pltpu.SemaphoreType.DMA,
        pltpu.SemaphoreType.DMA,
    ],
)
def _sc_update(items, A, B, t, newA, newB, out,
               idx_v, a_v, b_v, d_v, o_v, tb_v, t_v, g_sem, s_sem):
    wid = lax.axis_index("s") * NC + lax.axis_index("c")
    base = wid * PER_W

    for j in range(ROWS):
        pltpu.sync_copy(items.at[pl.ds(base + j * ROW, ROW)], idx_v.at[j])
    pltpu.sync_copy(t, t_v.at[pl.ds(0, 1)])

    gathers = []
    for j in range(ROWS):
        gathers.append(pltpu.async_copy(A.at[idx_v.at[j]], a_v.at[j], g_sem))
        gathers.append(pltpu.async_copy(B.at[idx_v.at[j]], b_v.at[j], g_sem))
    for c in gathers:
        c.wait()

    tn = t_v[...][0] + 1.0
    tnv = jnp.full((L,), tn, jnp.float32)
    for k in range(ROW // L):
        tb_v[pl.ds(k * L, L)] = tnv

    for j in range(ROWS):
        for k in range(ROW // L):
            sl = pl.ds(k * L, L)
            a = a_v[j, sl]
            b = b_v[j, sl]
            delta = (1.0 - ALPHA) * b + ALPHA * (tnv - a)
            d_v[j, sl] = delta
            bits = lax.bitcast_convert_type(delta, jnp.int32)
            e = lax.shift_right_logical(bits, 23) - 127
            m = lax.bitcast_convert_type(
                (bits & 0x007FFFFF) | 0x3F800000, jnp.float32)
            big = m >= _SQRT2
            m = jnp.where(big, 0.5 * m, m)
            ef = e.astype(jnp.float32) + jnp.where(big, 1.0, 0.0)
            s = (m - 1.0) / (m + 1.0)
            s2 = s * s
            p = s * (2.0 + s2 * (0.66666667 + s2 * (0.4 + s2 * 0.28571429)))
            o_v[j, sl] = 0.0 - (ef * _LN2 + p)

    scatters = []
    for j in range(ROWS):
        scatters.append(pltpu.async_copy(d_v.at[j], newB.at[idx_v.at[j]], s_sem))
        scatters.append(pltpu.async_copy(tb_v, newA.at[idx_v.at[j]], s_sem))
    for j in range(ROWS):
        pltpu.sync_copy(o_v.at[j], out.at[pl.ds(base + j * ROW, ROW)])
    for c in scatters:
        c.wait()


def kernel(items, A, B, t):
    t_new = t + 1.0
    newA = jax.new_ref(A)
    newB = jax.new_ref(B)
    out = _sc_update(items, A, B, t, newA, newB)
    return (out, newB[...], newA[...], t_new)


# speedup vs baseline: 2.6179x; 1.0120x over previous
import functools

import jax
import jax.numpy as jnp
from jax import lax
from jax.experimental import pallas as pl
from jax.experimental.pallas import tpu as pltpu
from jax.experimental.pallas import tpu_sc as plsc

ITEM_COUNT = 16384
ALPHA = 0.0001
NC = 2
NS = 16
L = 16
NW = NC * NS
PER_W = ITEM_COUNT // NW
ROW = 128
ROWS = PER_W // ROW

_LN2 = 0.6931471805599453
_SQRT2 = 1.4142135623730951

_mesh = plsc.VectorSubcoreMesh(
    core_axis_name="c", subcore_axis_name="s", num_cores=NC, num_subcores=NS
)


@functools.partial(
    pl.kernel,
    out_type=jax.ShapeDtypeStruct((ITEM_COUNT // ROW, ROW), jnp.float32),
    mesh=_mesh,
    scratch_types=[
        pltpu.VMEM((ROWS, ROW), jnp.int32),
        pltpu.VMEM((ROWS, ROW), jnp.float32),
        pltpu.VMEM((ROWS, ROW), jnp.float32),
        pltpu.VMEM((ROWS, ROW), jnp.float32),
        pltpu.VMEM((ROWS, ROW), jnp.float32),
        pltpu.VMEM((ROW,), jnp.float32),
        pltpu.SemaphoreType.DMA,
        pltpu.SemaphoreType.DMA,
    ],
)
def _sc_update(items2, A, B, tb, newA, newB, out,
               idx_v, a_v, b_v, d_v, o_v, tb_v, g_sem, s_sem):
    wid = lax.axis_index("s") * NC + lax.axis_index("c")
    rbase = wid * ROWS

    st_i = pltpu.async_copy(items2.at[pl.ds(rbase, ROWS)], idx_v, g_sem)
    st_t = pltpu.async_copy(tb, tb_v, g_sem)
    st_i.wait()
    st_t.wait()

    gathers = []
    for j in range(ROWS):
        gathers.append(pltpu.async_copy(A.at[idx_v.at[j]], a_v.at[j], g_sem))
        gathers.append(pltpu.async_copy(B.at[idx_v.at[j]], b_v.at[j], g_sem))
    for c in gathers:
        c.wait()

    tnv = tb_v[pl.ds(0, L)]
    for j in range(ROWS):
        for k in range(ROW // L):
            sl = pl.ds(k * L, L)
            a = a_v[j, sl]
            b = b_v[j, sl]
            delta = (1.0 - ALPHA) * b + ALPHA * (tnv - a)
            d_v[j, sl] = delta
            bits = lax.bitcast_convert_type(delta, jnp.int32)
            e = lax.shift_right_logical(bits, 23) - 127
            m = lax.bitcast_convert_type(
                (bits & 0x007FFFFF) | 0x3F800000, jnp.float32)
            big = m >= _SQRT2
            m = jnp.where(big, 0.5 * m, m)
            ef = e.astype(jnp.float32) + jnp.where(big, 1.0, 0.0)
            s = (m - 1.0) / (m + 1.0)
            s2 = s * s
            p = s * (2.0 + s2 * (0.66666667 + s2 * (0.4 + s2 * 0.28571429)))
            o_v[j, sl] = 0.0 - (ef * _LN2 + p)

    scatters = []
    for j in range(ROWS):
        scatters.append(pltpu.async_copy(d_v.at[j], newB.at[idx_v.at[j]], s_sem))
        scatters.append(pltpu.async_copy(tb_v, newA.at[idx_v.at[j]], s_sem))
    pltpu.sync_copy(o_v, out.at[pl.ds(rbase, ROWS)])
    for c in scatters:
        c.wait()


def kernel(items, A, B, t):
    t_new = t + 1.0
    items2 = items.reshape(ITEM_COUNT // ROW, ROW)
    tb = jnp.broadcast_to(t_new, (ROW,))
    newA = jax.new_ref(A)
    newB = jax.new_ref(B)
    out2 = _sc_update(items2, A, B, tb, newA, newB)
    return (out2.reshape(ITEM_COUNT), newB[...], newA[...], t_new)


# speedup vs baseline: 4.6603x; 1.7801x over previous
import functools

import jax
import jax.numpy as jnp
from jax import lax
from jax.experimental import pallas as pl
from jax.experimental.pallas import tpu as pltpu
from jax.experimental.pallas import tpu_sc as plsc

ITEM_COUNT = 16384
TABLE = 1000001
ALPHA = 0.0001
NC = 2
NS = 16
L = 16
ROW = 128
ROWS_T = ITEM_COUNT // (NS * ROW)

CHUNK = 62504
TAIL = TABLE - 15 * CHUNK
SH = 1000064

_LN2 = 0.6931471805599453
_SQRT2 = 1.4142135623730951

_mesh = plsc.VectorSubcoreMesh(
    core_axis_name="c", subcore_axis_name="s", num_cores=NC, num_subcores=NS
)


@functools.partial(
    pl.kernel,
    out_type=(
        jax.ShapeDtypeStruct((ITEM_COUNT // ROW, ROW), jnp.float32),
        jax.ShapeDtypeStruct((TABLE,), jnp.float32),
        jax.ShapeDtypeStruct((TABLE,), jnp.float32),
    ),
    mesh=_mesh,
    scratch_types=[
        pltpu.VMEM((ROWS_T, ROW), jnp.int32),
        pltpu.VMEM((ROWS_T, ROW), jnp.float32),
        pltpu.VMEM((ROWS_T, ROW), jnp.float32),
        pltpu.VMEM((ROWS_T, ROW), jnp.float32),
        pltpu.VMEM((ROWS_T, ROW), jnp.float32),
        pltpu.VMEM((ROW,), jnp.float32),
        pltpu.VMEM((CHUNK,), jnp.float32),
        pltpu.VMEM_SHARED((SH,), jnp.float32),
        pltpu.SemaphoreType.DMA,
        pltpu.SemaphoreType.DMA,
        pltpu.SemaphoreType.DMA,
    ],
)
def _sc_update(items2, A, B, tb, out, newB, newA,
               idx_v, a_v, b_v, d_v, o_v, tb_v, bounce, sh,
               f_sem, g_sem, s_sem):
    c = lax.axis_index("c")
    s = lax.axis_index("s")
    off = s * CHUNK
    is_c0 = c == 0
    is_tail = s == NS - 1

    @pl.when(jnp.logical_and(is_c0, jnp.logical_not(is_tail)))
    def _():
        pltpu.async_copy(A.at[pl.ds(off, CHUNK)], bounce, f_sem)

    @pl.when(jnp.logical_and(is_c0, is_tail))
    def _():
        pltpu.async_copy(A.at[pl.ds(off, TAIL)],
                         bounce.at[pl.ds(0, TAIL)], f_sem)

    @pl.when(jnp.logical_and(jnp.logical_not(is_c0), jnp.logical_not(is_tail)))
    def _():
        pltpu.async_copy(B.at[pl.ds(off, CHUNK)], bounce, f_sem)

    @pl.when(jnp.logical_and(jnp.logical_not(is_c0), is_tail))
    def _():
        pltpu.async_copy(B.at[pl.ds(off, TAIL)],
                         bounce.at[pl.ds(0, TAIL)], f_sem)

    st_i = pltpu.async_copy(items2.at[pl.ds(s * ROWS_T, ROWS_T)], idx_v, g_sem)
    st_t = pltpu.async_copy(tb, tb_v, g_sem)
    st_i.wait()
    st_t.wait()
    gathers = []
    for j in range(ROWS_T):
        gathers.append(pltpu.async_copy(A.at[idx_v.at[j]], a_v.at[j], g_sem))
        gathers.append(pltpu.async_copy(B.at[idx_v.at[j]], b_v.at[j], g_sem))

    @pl.when(jnp.logical_not(is_tail))
    def _():
        pltpu.make_async_copy(A.at[pl.ds(off, CHUNK)], bounce, f_sem).wait()

    @pl.when(is_tail)
    def _():
        pltpu.make_async_copy(A.at[pl.ds(off, TAIL)],
                              bounce.at[pl.ds(0, TAIL)], f_sem).wait()

    @pl.when(jnp.logical_not(is_tail))
    def _():
        pltpu.async_copy(bounce, sh.at[pl.ds(off, CHUNK)], f_sem)

    @pl.when(is_tail)
    def _():
        pltpu.async_copy(bounce.at[pl.ds(0, TAIL)],
                         sh.at[pl.ds(off, TAIL)], f_sem)

    for g in gathers:
        g.wait()

    tnv = tb_v[pl.ds(0, L)]

    def _row(j, carry):
        for k in range(ROW // L):
            sl = pl.ds(k * L, L)
            a = a_v[j, sl]
            b = b_v[j, sl]
            delta = (1.0 - ALPHA) * b + ALPHA * (tnv - a)
            d_v[j, sl] = jnp.where(is_c0, tnv, delta)
            bits = lax.bitcast_convert_type(delta, jnp.int32)
            e = lax.shift_right_logical(bits, 23) - 127
            m = lax.bitcast_convert_type(
                (bits & 0x007FFFFF) | 0x3F800000, jnp.float32)
            big = m >= _SQRT2
            m = jnp.where(big, 0.5 * m, m)
            ef = e.astype(jnp.float32) + jnp.where(big, 1.0, 0.0)
            sq = (m - 1.0) / (m + 1.0)
            s2 = sq * sq
            p = sq * (2.0 + s2 * (0.66666667 + s2 * (0.4 + s2 * 0.28571429)))
            o_v[j, sl] = 0.0 - (ef * _LN2 + p)
        return carry

    lax.fori_loop(0, ROWS_T, _row, 0)

    @pl.when(is_c0)
    def _():
        pltpu.sync_copy(o_v, out.at[pl.ds(s * ROWS_T, ROWS_T)])

    @pl.when(jnp.logical_not(is_tail))
    def _():
        pltpu.make_async_copy(bounce, sh.at[pl.ds(off, CHUNK)], f_sem).wait()

    @pl.when(is_tail)
    def _():
        pltpu.make_async_copy(bounce.at[pl.ds(0, TAIL)],
                              sh.at[pl.ds(off, TAIL)], f_sem).wait()

    plsc.subcore_barrier()

    scatters = []
    for j in range(ROWS_T):
        scatters.append(
            pltpu.async_copy(d_v.at[j], sh.at[idx_v.at[j]], s_sem))
    for sc in scatters:
        sc.wait()
    plsc.subcore_barrier()

    @pl.when(jnp.logical_not(is_tail))
    def _():
        pltpu.sync_copy(sh.at[pl.ds(off, CHUNK)], bounce)

    @pl.when(is_tail)
    def _():
        pltpu.sync_copy(sh.at[pl.ds(off, TAIL)], bounce.at[pl.ds(0, TAIL)])

    @pl.when(jnp.logical_and(is_c0, jnp.logical_not(is_tail)))
    def _():
        pltpu.async_copy(bounce, newA.at[pl.ds(off, CHUNK)], f_sem)

    @pl.when(jnp.logical_and(is_c0, is_tail))
    def _():
        pltpu.async_copy(bounce.at[pl.ds(0, TAIL)],
                         newA.at[pl.ds(off, TAIL)], f_sem)

    @pl.when(jnp.logical_and(jnp.logical_not(is_c0), jnp.logical_not(is_tail)))
    def _():
        pltpu.async_copy(bounce, newB.at[pl.ds(off, CHUNK)], f_sem)

    @pl.when(jnp.logical_and(jnp.logical_not(is_c0), is_tail))
    def _():
        pltpu.async_copy(bounce.at[pl.ds(0, TAIL)],
                         newB.at[pl.ds(off, TAIL)], f_sem)

    @pl.when(jnp.logical_not(is_tail))
    def _():
        pltpu.make_async_copy(bounce, newA.at[pl.ds(off, CHUNK)], f_sem).wait()

    @pl.when(is_tail)
    def _():
        pltpu.make_async_copy(bounce.at[pl.ds(0, TAIL)],
                              newA.at[pl.ds(off, TAIL)], f_sem).wait()


def kernel(items, A, B, t):
    t_new = t + 1.0
    items2 = items.reshape(ITEM_COUNT // ROW, ROW)
    tb = jnp.broadcast_to(t_new, (ROW,))
    out2, new_B, new_A = _sc_update(items2, A, B, tb)
    return (out2.reshape(ITEM_COUNT), new_B, new_A, t_new)


# speedup vs baseline: 4.7464x; 1.0185x over previous
import functools

import jax
import jax.numpy as jnp
from jax import lax
from jax.experimental import pallas as pl
from jax.experimental.pallas import tpu as pltpu
from jax.experimental.pallas import tpu_sc as plsc

ITEM_COUNT = 16384
TABLE = 1000001
ALPHA = 0.0001
NC = 2
NS = 16
L = 16
ROW = 128
ROWS_T = ITEM_COUNT // (NS * ROW)

CHUNK = 62504
TAIL = TABLE - 15 * CHUNK
SH = 1000064
WB0 = 31256

_LN2 = 0.6931471805599453
_SQRT2 = 1.4142135623730951

_mesh = plsc.VectorSubcoreMesh(
    core_axis_name="c", subcore_axis_name="s", num_cores=NC, num_subcores=NS
)


@functools.partial(
    pl.kernel,
    out_type=(
        jax.ShapeDtypeStruct((ITEM_COUNT // ROW, ROW), jnp.float32),
        jax.ShapeDtypeStruct((TABLE,), jnp.float32),
        jax.ShapeDtypeStruct((TABLE,), jnp.float32),
    ),
    mesh=_mesh,
    scratch_types=[
        pltpu.VMEM((ROWS_T, ROW), jnp.int32),
        pltpu.VMEM((ROWS_T, ROW), jnp.float32),
        pltpu.VMEM((ROWS_T, ROW), jnp.float32),
        pltpu.VMEM((ROWS_T, ROW), jnp.float32),
        pltpu.VMEM((ROWS_T, ROW), jnp.float32),
        pltpu.VMEM((ROW,), jnp.float32),
        pltpu.VMEM((CHUNK,), jnp.float32),
        pltpu.VMEM_SHARED((SH,), jnp.float32),
        pltpu.SemaphoreType.DMA,
        pltpu.SemaphoreType.DMA,
        pltpu.SemaphoreType.DMA,
    ],
)
def _sc_update(items2, A, B, tb, out, newB, newA,
               idx_v, a_v, b_v, d_v, o_v, tb_v, bounce, sh,
               f_sem, g_sem, s_sem):
    c = lax.axis_index("c")
    s = lax.axis_index("s")
    off = s * CHUNK
    is_c0 = c == 0
    is_tail = s == NS - 1

    @pl.when(jnp.logical_and(is_c0, jnp.logical_not(is_tail)))
    def _():
        pltpu.async_copy(A.at[pl.ds(off, CHUNK)], bounce, f_sem)

    @pl.when(jnp.logical_and(is_c0, is_tail))
    def _():
        pltpu.async_copy(A.at[pl.ds(off, TAIL)],
                         bounce.at[pl.ds(0, TAIL)], f_sem)

    @pl.when(jnp.logical_and(jnp.logical_not(is_c0), jnp.logical_not(is_tail)))
    def _():
        pltpu.async_copy(B.at[pl.ds(off, CHUNK)], bounce, f_sem)

    @pl.when(jnp.logical_and(jnp.logical_not(is_c0), is_tail))
    def _():
        pltpu.async_copy(B.at[pl.ds(off, TAIL)],
                         bounce.at[pl.ds(0, TAIL)], f_sem)

    st_i = pltpu.async_copy(items2.at[pl.ds(s * ROWS_T, ROWS_T)], idx_v, g_sem)
    st_t = pltpu.async_copy(tb, tb_v, g_sem)
    st_i.wait()
    st_t.wait()
    gathers = []
    for j in range(ROWS_T):
        gathers.append(pltpu.async_copy(A.at[idx_v.at[j]], a_v.at[j], g_sem))
        gathers.append(pltpu.async_copy(B.at[idx_v.at[j]], b_v.at[j], g_sem))

    @pl.when(jnp.logical_not(is_tail))
    def _():
        pltpu.make_async_copy(A.at[pl.ds(off, CHUNK)], bounce, f_sem).wait()

    @pl.when(is_tail)
    def _():
        pltpu.make_async_copy(A.at[pl.ds(off, TAIL)],
                              bounce.at[pl.ds(0, TAIL)], f_sem).wait()

    @pl.when(jnp.logical_not(is_tail))
    def _():
        pltpu.async_copy(bounce, sh.at[pl.ds(off, CHUNK)], f_sem)

    @pl.when(is_tail)
    def _():
        pltpu.async_copy(bounce.at[pl.ds(0, TAIL)],
                         sh.at[pl.ds(off, TAIL)], f_sem)

    for g in gathers:
        g.wait()

    tnv = tb_v[pl.ds(0, L)]

    def _row(j, carry):
        for k in range(ROW // L):
            sl = pl.ds(k * L, L)
            a = a_v[j, sl]
            b = b_v[j, sl]
            delta = (1.0 - ALPHA) * b + ALPHA * (tnv - a)
            d_v[j, sl] = jnp.where(is_c0, tnv, delta)
            bits = lax.bitcast_convert_type(delta, jnp.int32)
            e = lax.shift_right_logical(bits, 23) - 127
            m = lax.bitcast_convert_type(
                (bits & 0x007FFFFF) | 0x3F800000, jnp.float32)
            big = m >= _SQRT2
            m = jnp.where(big, 0.5 * m, m)
            ef = e.astype(jnp.float32) + jnp.where(big, 1.0, 0.0)
            sq = (m - 1.0) / (m + 1.0)
            s2 = sq * sq
            p = sq * (2.0 + s2 * (0.66666667 + s2 * (0.4 + s2 * 0.28571429)))
            o_v[j, sl] = 0.0 - (ef * _LN2 + p)
        return carry

    lax.fori_loop(0, ROWS_T, _row, 0)

    @pl.when(is_c0)
    def _():
        pltpu.sync_copy(o_v, out.at[pl.ds(s * ROWS_T, ROWS_T)])

    @pl.when(jnp.logical_not(is_tail))
    def _():
        pltpu.make_async_copy(bounce, sh.at[pl.ds(off, CHUNK)], f_sem).wait()

    @pl.when(is_tail)
    def _():
        pltpu.make_async_copy(bounce.at[pl.ds(0, TAIL)],
                              sh.at[pl.ds(off, TAIL)], f_sem).wait()

    plsc.subcore_barrier()

    scatters = []
    for j in range(ROWS_T):
        scatters.append(
            pltpu.async_copy(d_v.at[j], sh.at[idx_v.at[j]], s_sem))
    for sc in scatters:
        sc.wait()
    plsc.subcore_barrier()

    for (hoff, hlen_n, hlen_t) in ((0, WB0, WB0), (WB0, CHUNK - WB0, TAIL - WB0)):
        @pl.when(jnp.logical_not(is_tail))
        def _(hoff=hoff, hlen=hlen_n):
            pltpu.sync_copy(sh.at[pl.ds(off + hoff, hlen)],
                            bounce.at[pl.ds(hoff, hlen)])

        @pl.when(is_tail)
        def _(hoff=hoff, hlen=hlen_t):
            pltpu.sync_copy(sh.at[pl.ds(off + hoff, hlen)],
                            bounce.at[pl.ds(hoff, hlen)])

        @pl.when(jnp.logical_and(is_c0, jnp.logical_not(is_tail)))
        def _(hoff=hoff, hlen=hlen_n):
            pltpu.async_copy(bounce.at[pl.ds(hoff, hlen)],
                             newA.at[pl.ds(off + hoff, hlen)], f_sem)

        @pl.when(jnp.logical_and(is_c0, is_tail))
        def _(hoff=hoff, hlen=hlen_t):
            pltpu.async_copy(bounce.at[pl.ds(hoff, hlen)],
                             newA.at[pl.ds(off + hoff, hlen)], f_sem)

        @pl.when(jnp.logical_and(jnp.logical_not(is_c0),
                                 jnp.logical_not(is_tail)))
        def _(hoff=hoff, hlen=hlen_n):
            pltpu.async_copy(bounce.at[pl.ds(hoff, hlen)],
                             newB.at[pl.ds(off + hoff, hlen)], f_sem)

        @pl.when(jnp.logical_and(jnp.logical_not(is_c0), is_tail))
        def _(hoff=hoff, hlen=hlen_t):
            pltpu.async_copy(bounce.at[pl.ds(hoff, hlen)],
                             newB.at[pl.ds(off + hoff, hlen)], f_sem)

    @pl.when(jnp.logical_not(is_tail))
    def _():
        pltpu.make_async_copy(bounce, newA.at[pl.ds(off, CHUNK)], f_sem).wait()

    @pl.when(is_tail)
    def _():
        pltpu.make_async_copy(bounce.at[pl.ds(0, TAIL)],
                              newA.at[pl.ds(off, TAIL)], f_sem).wait()


def kernel(items, A, B, t):
    t_new = t + 1.0
    items2 = items.reshape(ITEM_COUNT // ROW, ROW)
    tb = jnp.broadcast_to(t_new, (ROW,))
    out2, new_B, new_A = _sc_update(items2, A, B, tb)
    return (out2.reshape(ITEM_COUNT), new_B, new_A, t_new)


# speedup vs baseline: 4.9030x; 1.0330x over previous
import functools

import jax
import jax.numpy as jnp
from jax import lax
from jax.experimental import pallas as pl
from jax.experimental.pallas import tpu as pltpu
from jax.experimental.pallas import tpu_sc as plsc

ITEM_COUNT = 16384
TABLE = 1000001
ALPHA = 0.0001
NC = 2
NS = 16
L = 16
ROW = 128
ROWS_T = ITEM_COUNT // (NS * ROW)

CHUNK = 62504
TAIL = TABLE - 15 * CHUNK
SH = 1000064
WB0 = 31256

_LN2 = 0.6931471805599453
_SQRT2 = 1.4142135623730951

_mesh = plsc.VectorSubcoreMesh(
    core_axis_name="c", subcore_axis_name="s", num_cores=NC, num_subcores=NS
)


@functools.partial(
    pl.kernel,
    out_type=(
        jax.ShapeDtypeStruct((ITEM_COUNT // ROW, ROW), jnp.float32),
        jax.ShapeDtypeStruct((TABLE,), jnp.float32),
        jax.ShapeDtypeStruct((TABLE,), jnp.float32),
    ),
    mesh=_mesh,
    scratch_types=[
        pltpu.VMEM((ROWS_T, ROW), jnp.int32),
        pltpu.VMEM((ROWS_T, ROW), jnp.float32),
        pltpu.VMEM((ROWS_T, ROW), jnp.float32),
        pltpu.VMEM((ROWS_T, ROW), jnp.float32),
        pltpu.VMEM((ROWS_T, ROW), jnp.float32),
        pltpu.VMEM((ROW,), jnp.float32),
        pltpu.VMEM((CHUNK,), jnp.float32),
        pltpu.VMEM_SHARED((SH,), jnp.float32),
        pltpu.SemaphoreType.DMA,
        pltpu.SemaphoreType.DMA,
        pltpu.SemaphoreType.DMA,
    ],
)
def _sc_update(items2, A, B, tb, out, newB, newA,
               idx_v, a_v, b_v, d_v, o_v, tb_v, bounce, sh,
               f_sem, g_sem, s_sem):
    c = lax.axis_index("c")
    s = lax.axis_index("s")
    off = s * CHUNK
    is_c0 = c == 0
    is_tail = s == NS - 1

    @pl.when(jnp.logical_and(is_c0, jnp.logical_not(is_tail)))
    def _():
        pltpu.async_copy(A.at[pl.ds(off, CHUNK)], bounce, f_sem)

    @pl.when(jnp.logical_and(is_c0, is_tail))
    def _():
        pltpu.async_copy(A.at[pl.ds(off, TAIL)],
                         bounce.at[pl.ds(0, TAIL)], f_sem)

    @pl.when(jnp.logical_and(jnp.logical_not(is_c0), jnp.logical_not(is_tail)))
    def _():
        pltpu.async_copy(B.at[pl.ds(off, CHUNK)], bounce, f_sem)

    @pl.when(jnp.logical_and(jnp.logical_not(is_c0), is_tail))
    def _():
        pltpu.async_copy(B.at[pl.ds(off, TAIL)],
                         bounce.at[pl.ds(0, TAIL)], f_sem)

    st_i = pltpu.async_copy(items2.at[pl.ds(s * ROWS_T, ROWS_T)], idx_v, g_sem)
    st_t = pltpu.async_copy(tb, tb_v, g_sem)
    st_i.wait()
    st_t.wait()
    @pl.when(jnp.logical_not(is_c0))
    def _():
        for j in range(ROWS_T):
            pltpu.async_copy(A.at[idx_v.at[j]], a_v.at[j], g_sem)
            pltpu.async_copy(B.at[idx_v.at[j]], b_v.at[j], g_sem)

    @pl.when(jnp.logical_not(is_tail))
    def _():
        pltpu.make_async_copy(A.at[pl.ds(off, CHUNK)], bounce, f_sem).wait()

    @pl.when(is_tail)
    def _():
        pltpu.make_async_copy(A.at[pl.ds(off, TAIL)],
                              bounce.at[pl.ds(0, TAIL)], f_sem).wait()

    @pl.when(jnp.logical_not(is_tail))
    def _():
        pltpu.async_copy(bounce, sh.at[pl.ds(off, CHUNK)], f_sem)

    @pl.when(is_tail)
    def _():
        pltpu.async_copy(bounce.at[pl.ds(0, TAIL)],
                         sh.at[pl.ds(off, TAIL)], f_sem)

    tnv = tb_v[pl.ds(0, L)]

    @pl.when(jnp.logical_not(is_c0))
    def _():
        for j in range(ROWS_T):
            pltpu.make_async_copy(A.at[idx_v.at[j]], a_v.at[j], g_sem).wait()
            pltpu.make_async_copy(B.at[idx_v.at[j]], b_v.at[j], g_sem).wait()

        def _row(j, carry):
            for k in range(ROW // L):
                sl = pl.ds(k * L, L)
                a = a_v[j, sl]
                b = b_v[j, sl]
                delta = (1.0 - ALPHA) * b + ALPHA * (tnv - a)
                d_v[j, sl] = delta
                bits = lax.bitcast_convert_type(delta, jnp.int32)
                e = lax.shift_right_logical(bits, 23) - 127
                m = lax.bitcast_convert_type(
                    (bits & 0x007FFFFF) | 0x3F800000, jnp.float32)
                big = m >= _SQRT2
                m = jnp.where(big, 0.5 * m, m)
                ef = e.astype(jnp.float32) + jnp.where(big, 1.0, 0.0)
                sq = (m - 1.0) / (m + 1.0)
                s2 = sq * sq
                p = sq * (2.0 + s2 * (0.66666667 + s2 * (0.4 + s2 * 0.28571429)))
                o_v[j, sl] = 0.0 - (ef * _LN2 + p)
            return carry

        lax.fori_loop(0, ROWS_T, _row, 0)
        pltpu.sync_copy(o_v, out.at[pl.ds(s * ROWS_T, ROWS_T)])

    @pl.when(jnp.logical_not(is_tail))
    def _():
        pltpu.make_async_copy(bounce, sh.at[pl.ds(off, CHUNK)], f_sem).wait()

    @pl.when(is_tail)
    def _():
        pltpu.make_async_copy(bounce.at[pl.ds(0, TAIL)],
                              sh.at[pl.ds(off, TAIL)], f_sem).wait()

    plsc.subcore_barrier()

    @pl.when(is_c0)
    def _():
        for j in range(ROWS_T):
            pltpu.async_copy(tb_v, sh.at[idx_v.at[j]], s_sem)

    @pl.when(jnp.logical_not(is_c0))
    def _():
        for j in range(ROWS_T):
            pltpu.async_copy(d_v.at[j], sh.at[idx_v.at[j]], s_sem)

    for j in range(ROWS_T):
        pltpu.make_async_copy(d_v.at[j], sh.at[idx_v.at[j]], s_sem).wait()
    plsc.subcore_barrier()

    for (hoff, hlen_n, hlen_t) in ((0, WB0, WB0), (WB0, CHUNK - WB0, TAIL - WB0)):
        @pl.when(jnp.logical_not(is_tail))
        def _(hoff=hoff, hlen=hlen_n):
            pltpu.sync_copy(sh.at[pl.ds(off + hoff, hlen)],
                            bounce.at[pl.ds(hoff, hlen)])

        @pl.when(is_tail)
        def _(hoff=hoff, hlen=hlen_t):
            pltpu.sync_copy(sh.at[pl.ds(off + hoff, hlen)],
                            bounce.at[pl.ds(hoff, hlen)])

        @pl.when(jnp.logical_and(is_c0, jnp.logical_not(is_tail)))
        def _(hoff=hoff, hlen=hlen_n):
            pltpu.async_copy(bounce.at[pl.ds(hoff, hlen)],
                             newA.at[pl.ds(off + hoff, hlen)], f_sem)

        @pl.when(jnp.logical_and(is_c0, is_tail))
        def _(hoff=hoff, hlen=hlen_t):
            pltpu.async_copy(bounce.at[pl.ds(hoff, hlen)],
                             newA.at[pl.ds(off + hoff, hlen)], f_sem)

        @pl.when(jnp.logical_and(jnp.logical_not(is_c0),
                                 jnp.logical_not(is_tail)))
        def _(hoff=hoff, hlen=hlen_n):
            pltpu.async_copy(bounce.at[pl.ds(hoff, hlen)],
                             newB.at[pl.ds(off + hoff, hlen)], f_sem)

        @pl.when(jnp.logical_and(jnp.logical_not(is_c0), is_tail))
        def _(hoff=hoff, hlen=hlen_t):
            pltpu.async_copy(bounce.at[pl.ds(hoff, hlen)],
                             newB.at[pl.ds(off + hoff, hlen)], f_sem)

    @pl.when(jnp.logical_not(is_tail))
    def _():
        pltpu.make_async_copy(bounce, newA.at[pl.ds(off, CHUNK)], f_sem).wait()

    @pl.when(is_tail)
    def _():
        pltpu.make_async_copy(bounce.at[pl.ds(0, TAIL)],
                              newA.at[pl.ds(off, TAIL)], f_sem).wait()


def kernel(items, A, B, t):
    t_new = t + 1.0
    items2 = items.reshape(ITEM_COUNT // ROW, ROW)
    tb = jnp.broadcast_to(t_new, (ROW,))
    out2, new_B, new_A = _sc_update(items2, A, B, tb)
    return (out2.reshape(ITEM_COUNT), new_B, new_A, t_new)
